# baseline (speedup 1.0000x reference)
import functools

import jax
import jax.numpy as jnp
from jax import lax
from jax.experimental import pallas as pl
from jax.experimental.pallas import tpu as pltpu
from jax.experimental.pallas import tpu_sc as plsc

V_DIM = 1_000_000
EMB = 64
BATCH = 16384
CTX = 20
LANES = 16
ROW_W = 128

NC = 2
NS = 16
NW = NC * NS

B_PER_W = BATCH // NW
T = 128
NCHUNK = B_PER_W // T


def _cbow_body(
    xt_hbm, tbl_hbm, out_hbm, xt_v, acc0_v, acc1_v, outc0_v, outc1_v,
    gsem0, gsem1, osem,
):
    wid = lax.axis_index("s") * NC + lax.axis_index("c")
    base = wid * B_PER_W
    pltpu.sync_copy(xt_hbm.at[:, pl.ds(base, B_PER_W)], xt_v)

    for c in range(CTX):
        for q in range(B_PER_W // LANES):
            sl = pl.ds(q * LANES, LANES)
            xt_v[c, sl] = lax.shift_left(xt_v[c, sl], 1)

    zero = lax.broadcast(jnp.bfloat16(0.0), (2 * LANES,))
    accs = [acc0_v, acc1_v]
    outcs = [outc0_v, outc1_v]
    gsems = [gsem0, gsem1]

    def zero_acc(acc_v):
        def z_body(rr, zcarry):
            for k in range(EMB // (2 * LANES)):
                acc_v[rr, pl.ds(k * 2 * LANES, 2 * LANES)] = zero
            return zcarry

        lax.fori_loop(0, T, z_body, 0)

    def fire(t, acc_v):
        return [
            pltpu.async_copy(
                tbl_hbm.at[xt_v.at[c, pl.ds(t * T, T)]],
                acc_v,
                gsems[t % 2],
                add=True,
            )
            for c in range(CTX)
        ]

    ev_idx = 2 * lax.iota(jnp.int32, LANES)
    od_idx = ev_idx + 1
    mask16 = lax.broadcast(jnp.int32(-65536), (LANES,))

    def scale_and_out(t, acc_v, outc_v):
        def s_body(rr, scarry):
            for q in range(EMB // (2 * LANES)):
                pair = plsc.bitcast(
                    acc_v[rr, pl.ds(q * 2 * LANES, 2 * LANES)], jnp.int32
                )
                ev = plsc.bitcast(
                    lax.shift_left(pair, 16), jnp.float32
                ) * (1.0 / CTX)
                od = plsc.bitcast(pair & mask16, jnp.float32) * (1.0 / CTX)
                b0 = q * 2 * LANES
                plsc.store_scatter(outc_v, [lax.broadcast(rr, (LANES,)), b0 + ev_idx], ev)
                plsc.store_scatter(outc_v, [lax.broadcast(rr, (LANES,)), b0 + od_idx], od)
            return scarry

        lax.fori_loop(0, T, s_body, 0)
        return pltpu.async_copy(
            outc_v, out_hbm.at[pl.ds(base + t * T, T), :], osem
        )

    zero_acc(accs[0])
    gathers = {0: fire(0, accs[0])}
    outs = {}
    for t in range(NCHUNK):
        nxt = t + 1
        if nxt < NCHUNK:
            if nxt >= 2:
                outs.pop(nxt - 2).wait()
            zero_acc(accs[nxt % 2])
            gathers[nxt] = fire(nxt, accs[nxt % 2])
        for cp in gathers.pop(t):
            cp.wait()
        outs[t] = scale_and_out(t, accs[t % 2], outcs[t % 2])
    for t in sorted(outs):
        outs.pop(t).wait()


def kernel(x, table):
    tblp = jnp.pad(table, ((0, 0), (0, ROW_W - EMB))).astype(jnp.bfloat16)
    tbl2 = tblp.reshape(2 * V_DIM, EMB)
    xt = x.T

    mesh = plsc.VectorSubcoreMesh(core_axis_name="c", subcore_axis_name="s")
    run = functools.partial(
        pl.kernel,
        mesh=mesh,
        out_type=jax.ShapeDtypeStruct((BATCH, EMB), jnp.float32),
        scratch_types=[
            pltpu.VMEM((CTX, B_PER_W), jnp.int32),
            pltpu.VMEM((T, EMB), jnp.bfloat16),
            pltpu.VMEM((T, EMB), jnp.bfloat16),
            pltpu.VMEM((T, EMB), jnp.float32),
            pltpu.VMEM((T, EMB), jnp.float32),
            pltpu.SemaphoreType.DMA,
            pltpu.SemaphoreType.DMA,
            pltpu.SemaphoreType.DMA,
        ],
        compiler_params=pltpu.CompilerParams(
            use_tc_tiling_on_sc=False, needs_layout_passes=False
        ),
    )(_cbow_body)
    return run(xt, tbl2)


# speedup vs baseline: 2.2134x; 2.2134x over previous
import functools

import jax
import jax.numpy as jnp
from jax import lax
from jax.experimental import pallas as pl
from jax.experimental.pallas import tpu as pltpu
from jax.experimental.pallas import tpu_sc as plsc

V_DIM = 1_000_000
EMB = 64
BATCH = 16384
CTX = 20
LANES = 16
ROW_W = 128

NC = 2
NS = 16
NW = NC * NS

B_PER_W = BATCH // NW
T = 128
NCHUNK = B_PER_W // T


def _cbow_body(xt_hbm, tbl_hbm, out_hbm, xt_v, acc0_v, acc1_v, gsem0, gsem1, osem):
    wid = lax.axis_index("s") * NC + lax.axis_index("c")
    base = wid * B_PER_W
    pltpu.sync_copy(xt_hbm.at[:, pl.ds(base, B_PER_W)], xt_v)

    for c in range(CTX):
        for q in range(B_PER_W // LANES):
            sl = pl.ds(q * LANES, LANES)
            xt_v[c, sl] = lax.shift_left(xt_v[c, sl], 1)

    zero = lax.broadcast(jnp.float32(0.0), (LANES,))
    accs = [acc0_v, acc1_v]
    gsems = [gsem0, gsem1]

    def zero_acc(acc_v):
        def z_body(rr, zcarry):
            for k in range(EMB // LANES):
                acc_v[rr, pl.ds(k * LANES, LANES)] = zero
            return zcarry

        lax.fori_loop(0, T, z_body, 0)

    def fire(t, acc_v):
        return [
            pltpu.async_copy(
                tbl_hbm.at[xt_v.at[c, pl.ds(t * T, T)]],
                acc_v,
                gsems[t % 2],
                add=True,
            )
            for c in range(CTX)
        ]

    def scale_and_out(t, acc_v):
        def s_body(rr, scarry):
            for k in range(EMB // LANES):
                sl = pl.ds(k * LANES, LANES)
                acc_v[rr, sl] = acc_v[rr, sl] * (1.0 / CTX)
            return scarry

        lax.fori_loop(0, T, s_body, 0)
        return pltpu.async_copy(
            acc_v, out_hbm.at[pl.ds(base + t * T, T), :], osem
        )

    zero_acc(accs[0])
    gathers = {0: fire(0, accs[0])}
    outs = {}
    for t in range(NCHUNK):
        nxt = t + 1
        if nxt < NCHUNK:
            if nxt >= 2:
                outs.pop(nxt - 2).wait()
            zero_acc(accs[nxt % 2])
            gathers[nxt] = fire(nxt, accs[nxt % 2])
        for cp in gathers.pop(t):
            cp.wait()
        outs[t] = scale_and_out(t, accs[t % 2])
    for t in sorted(outs):
        outs.pop(t).wait()


def kernel(x, table):
    tblp = jnp.pad(table, ((0, 0), (0, ROW_W - EMB)))
    tbl2 = tblp.reshape(2 * V_DIM, EMB)
    xt = x.T

    mesh = plsc.VectorSubcoreMesh(core_axis_name="c", subcore_axis_name="s")
    run = functools.partial(
        pl.kernel,
        mesh=mesh,
        out_type=jax.ShapeDtypeStruct((BATCH, EMB), jnp.float32),
        scratch_types=[
            pltpu.VMEM((CTX, B_PER_W), jnp.int32),
            pltpu.VMEM((T, EMB), jnp.float32),
            pltpu.VMEM((T, EMB), jnp.float32),
            pltpu.SemaphoreType.DMA,
            pltpu.SemaphoreType.DMA,
            pltpu.SemaphoreType.DMA,
        ],
        compiler_params=pltpu.CompilerParams(use_tc_tiling_on_sc=False),
    )(_cbow_body)
    return run(xt, tbl2)
